# Initial kernel scaffold; baseline (speedup 1.0000x reference)
import functools

import jax
import jax.numpy as jnp
from jax import lax
from jax.experimental import pallas as pl
from jax.experimental.pallas import tpu as pltpu
from jax.experimental.pallas import tpu_sc as plsc

_N = 10000
_E = 320000
_DI = 128
_DH = 256
_DA = 144
_EPS = 1e-5

_CORES = 2
_TILES = 16
_K = 80
_RPT = _N // _TILES


def _make_agg(D, mode):
  per_tile = _E // (_CORES * _TILES) if mode == "edge" else _E // _TILES
  chunks = per_tile // _K
  assert chunks * _K == per_tile

  mesh = plsc.VectorSubcoreMesh(core_axis_name="c", subcore_axis_name="s")

  @functools.partial(
      pl.kernel,
      mesh=mesh,
      out_type=jax.ShapeDtypeStruct((_CORES, _N, D), jnp.float32),
      scratch_types=[
          pltpu.VMEM_SHARED((_N, D), jnp.float32),
          pltpu.VMEM((_K,), jnp.int32),
          pltpu.VMEM((_K,), jnp.int32),
          pltpu.VMEM((_K, D), jnp.float32),
          pltpu.SemaphoreType.DMA,
      ],
  )
  def agg(table, src_hbm, dst_hbm, zeros_hbm, out_hbm, acc, srcb, dstb, rows,
          sem):
    c = lax.axis_index("c")
    s = lax.axis_index("s")
    pltpu.sync_copy(zeros_hbm.at[pl.ds(s * _RPT, _RPT)],
                    acc.at[pl.ds(s * _RPT, _RPT)])
    plsc.subcore_barrier()

    base = (c * _TILES + s) * per_tile if mode == "edge" else s * per_tile

    def chunk(i, carry):
      off = base + i * _K
      pltpu.sync_copy(src_hbm.at[pl.ds(off, _K)], srcb)
      pltpu.sync_copy(dst_hbm.at[pl.ds(off, _K)], dstb)
      if mode == "feat":
        roff = c * _N
        for j in range(_K // 16):
          srcb[pl.ds(j * 16, 16)] = srcb[pl.ds(j * 16, 16)] + roff
      pltpu.async_copy(table.at[srcb], rows, sem).wait()
      pltpu.sync_copy(rows, acc.at[dstb], add=True)
      return carry

    lax.fori_loop(0, chunks, chunk, 0)
    plsc.subcore_barrier()
    pltpu.sync_copy(acc.at[pl.ds(s * _RPT, _RPT)],
                    out_hbm.at[c, pl.ds(s * _RPT, _RPT)])

  return agg


_agg0 = _make_agg(_DA, "edge")
_agg1 = _make_agg(_DI, "feat")
_agg2 = _make_agg(_DI, "edge")


def _bn_relu(h, g, be):
  mu = jnp.mean(h, 0, keepdims=True)
  var = jnp.mean((h - mu) ** 2, 0, keepdims=True)
  return jnp.maximum((h - mu) * lax.rsqrt(var + _EPS) * g + be, 0.0)


def _mm_t(a, w):
  return lax.dot_general(a, w, (((1,), (1,)), ((), ())),
                         preferred_element_type=jnp.float32)


def _tc_a_body(a0_ref, x_ref, wl_ref, bl_ref, wr_ref, g_ref, be_ref,
               ht_ref, rinv_ref):
  a = a0_ref[...]
  s = a[0] + a[1]
  deg = s[:, _DI:_DI + 1]
  rinv = 1.0 / jnp.maximum(deg, 1.0)
  h = (_mm_t(s[:, :_DI] * rinv, wl_ref[...]) + bl_ref[...]
       + _mm_t(x_ref[...], wr_ref[...]))
  h1 = _bn_relu(h, g_ref[...], be_ref[...])
  ht_ref[...] = jnp.concatenate([h1[:, :_DI], h1[:, _DI:]], 0)
  rinv_ref[...] = rinv


def _tc_b_body(a1_ref, ht_ref, rinv_ref, wl_ref, bl_ref, wr_ref, g_ref,
               be_ref, wl2_ref, bl2_ref, wr2_ref, p_ref, r_ref):
  a = a1_ref[...]
  ht = ht_ref[...]
  rinv = rinv_ref[...]
  agg = jnp.concatenate([a[0], a[1]], 1) * rinv
  h1 = jnp.concatenate([ht[:_N], ht[_N:]], 1)
  h = _mm_t(agg, wl_ref[...]) + bl_ref[...] + _mm_t(h1, wr_ref[...])
  h2 = _bn_relu(h, g_ref[...], be_ref[...])
  p_ref[...] = _mm_t(h2, wl2_ref[...])
  r_ref[...] = _mm_t(h2, wr2_ref[...]) + bl2_ref[...]


def _tc_c_body(a2_ref, rinv_ref, r_ref, out_ref):
  a = a2_ref[...]
  h = (a[0] + a[1]) * rinv_ref[...] + r_ref[...]
  m = jnp.max(h, -1, keepdims=True)
  lse = jnp.log(jnp.sum(jnp.exp(h - m), -1, keepdims=True)) + m
  out_ref[...] = h - lse


def kernel(x, edge_index, Wl0, bl0, Wr0, Wl1, bl1, Wr1, Wl2, bl2, Wr2,
           g1, be1, g2, be2):
  src = edge_index[0]
  dst = edge_index[1]
  f32 = jnp.float32
  x_aug = jnp.concatenate(
      [x, jnp.ones((_N, 1), f32), jnp.zeros((_N, _DA - _DI - 1), f32)], 1)
  z144 = jnp.zeros((_N, _DA), f32)
  z128 = jnp.zeros((_N, _DI), f32)

  a0 = _agg0(x_aug, src, dst, z144)
  ht, rinv = pl.pallas_call(
      _tc_a_body,
      out_shape=[jax.ShapeDtypeStruct((2 * _N, _DI), f32),
                 jax.ShapeDtypeStruct((_N, 1), f32)],
  )(a0, x, Wl0, bl0.reshape(1, -1), Wr0, g1.reshape(1, -1),
    be1.reshape(1, -1))

  a1 = _agg1(ht, src, dst, z128)
  p, r = pl.pallas_call(
      _tc_b_body,
      out_shape=[jax.ShapeDtypeStruct((_N, _DI), f32),
                 jax.ShapeDtypeStruct((_N, _DI), f32)],
  )(a1, ht, rinv, Wl1, bl1.reshape(1, -1), Wr1, g2.reshape(1, -1),
    be2.reshape(1, -1), Wl2, bl2.reshape(1, -1), Wr2)

  a2 = _agg2(p, src, dst, z128)
  out = pl.pallas_call(
      _tc_c_body,
      out_shape=jax.ShapeDtypeStruct((_N, _DI), f32),
  )(a2, rinv, r)
  return out


# speedup vs baseline: 4.8099x; 4.8099x over previous
import functools

import jax
import jax.numpy as jnp
from jax import lax
from jax.experimental import pallas as pl
from jax.experimental.pallas import tpu as pltpu
from jax.experimental.pallas import tpu_sc as plsc

_N = 10000
_E = 320000
_DI = 128
_DH = 256
_DA = 144
_EPS = 1e-5

_CORES = 2
_TILES = 16
_K = 80
_NP = 10240
_RPT = _NP // _TILES


def _make_agg(D, mode):
  per_tile = _E // (_CORES * _TILES) if mode == "edge" else _E // _TILES
  chunks = per_tile // _K
  assert chunks * _K == per_tile

  mesh = plsc.VectorSubcoreMesh(core_axis_name="c", subcore_axis_name="s")

  @functools.partial(
      pl.kernel,
      mesh=mesh,
      out_type=jax.ShapeDtypeStruct((_CORES, _NP, D), jnp.float32),
      scratch_types=[
          pltpu.VMEM_SHARED((_NP, D), jnp.float32),
          pltpu.VMEM((_K,), jnp.int32),
          pltpu.VMEM((_K,), jnp.int32),
          pltpu.VMEM((_K, D), jnp.float32),
          pltpu.SemaphoreType.DMA,
      ],
      compiler_params=pltpu.CompilerParams(use_tc_tiling_on_sc=False),
  )
  def agg(table, src_hbm, dst_hbm, zeros_hbm, out_hbm, acc, srcb, dstb, rows,
          sem):
    c = lax.axis_index("c")
    s = lax.axis_index("s")
    pltpu.sync_copy(zeros_hbm.at[pl.ds(s * _RPT, _RPT)],
                    acc.at[pl.ds(s * _RPT, _RPT)])
    plsc.subcore_barrier()

    base = (c * _TILES + s) * per_tile if mode == "edge" else s * per_tile

    def chunk(i, carry):
      off = base + i * _K
      pltpu.sync_copy(src_hbm.at[pl.ds(off, _K)], srcb)
      pltpu.sync_copy(dst_hbm.at[pl.ds(off, _K)], dstb)
      if mode == "feat":
        roff = c * _N
        for j in range(_K // 16):
          srcb[pl.ds(j * 16, 16)] = srcb[pl.ds(j * 16, 16)] + roff
      pltpu.async_copy(table.at[srcb], rows, sem).wait()
      pltpu.sync_copy(rows, acc.at[dstb], add=True)
      return carry

    lax.fori_loop(0, chunks, chunk, 0)
    plsc.subcore_barrier()
    pltpu.sync_copy(acc.at[pl.ds(s * _RPT, _RPT)],
                    out_hbm.at[c, pl.ds(s * _RPT, _RPT)])

  return agg


_agg0 = _make_agg(_DA, "edge")
_agg1 = _make_agg(_DI, "feat")
_agg2 = _make_agg(_DI, "edge")


def _bn_relu(h, g, be):
  mu = jnp.mean(h, 0, keepdims=True)
  var = jnp.mean((h - mu) ** 2, 0, keepdims=True)
  return jnp.maximum((h - mu) * lax.rsqrt(var + _EPS) * g + be, 0.0)


def _mm_t(a, w):
  return lax.dot_general(a, w, (((1,), (1,)), ((), ())),
                         preferred_element_type=jnp.float32)


def _tc_a_body(a0_ref, x_ref, wl_ref, bl_ref, wr_ref, g_ref, be_ref,
               ht_ref, rinv_ref):
  a = a0_ref[...]
  s = a[0, :_N] + a[1, :_N]
  deg = s[:, _DI:_DI + 1]
  rinv = 1.0 / jnp.maximum(deg, 1.0)
  h = (_mm_t(s[:, :_DI] * rinv, wl_ref[...]) + bl_ref[...]
       + _mm_t(x_ref[...], wr_ref[...]))
  h1 = _bn_relu(h, g_ref[...], be_ref[...])
  ht_ref[...] = jnp.concatenate([h1[:, :_DI], h1[:, _DI:]], 0)
  rinv_ref[...] = rinv


def _tc_b_body(a1_ref, ht_ref, rinv_ref, wl_ref, bl_ref, wr_ref, g_ref,
               be_ref, wl2_ref, bl2_ref, wr2_ref, p_ref, r_ref):
  a = a1_ref[...]
  ht = ht_ref[...]
  rinv = rinv_ref[...]
  agg = jnp.concatenate([a[0, :_N], a[1, :_N]], 1) * rinv
  h1 = jnp.concatenate([ht[:_N], ht[_N:]], 1)
  h = _mm_t(agg, wl_ref[...]) + bl_ref[...] + _mm_t(h1, wr_ref[...])
  h2 = _bn_relu(h, g_ref[...], be_ref[...])
  p_ref[...] = _mm_t(h2, wl2_ref[...])
  r_ref[...] = _mm_t(h2, wr2_ref[...]) + bl2_ref[...]


def _tc_c_body(a2_ref, rinv_ref, r_ref, out_ref):
  a = a2_ref[...]
  h = (a[0, :_N] + a[1, :_N]) * rinv_ref[...] + r_ref[...]
  m = jnp.max(h, -1, keepdims=True)
  lse = jnp.log(jnp.sum(jnp.exp(h - m), -1, keepdims=True)) + m
  out_ref[...] = h - lse


def kernel(x, edge_index, Wl0, bl0, Wr0, Wl1, bl1, Wr1, Wl2, bl2, Wr2,
           g1, be1, g2, be2):
  src = edge_index[0]
  dst = edge_index[1]
  f32 = jnp.float32
  x_aug = jnp.concatenate(
      [x, jnp.ones((_N, 1), f32), jnp.zeros((_N, _DA - _DI - 1), f32)], 1)
  z144 = jnp.zeros((_NP, _DA), f32)
  z128 = jnp.zeros((_NP, _DI), f32)

  a0 = _agg0(x_aug, src, dst, z144)
  ht, rinv = pl.pallas_call(
      _tc_a_body,
      out_shape=[jax.ShapeDtypeStruct((2 * _N, _DI), f32),
                 jax.ShapeDtypeStruct((_N, 1), f32)],
  )(a0, x, Wl0, bl0.reshape(1, -1), Wr0, g1.reshape(1, -1),
    be1.reshape(1, -1))

  a1 = _agg1(ht, src, dst, z128)
  p, r = pl.pallas_call(
      _tc_b_body,
      out_shape=[jax.ShapeDtypeStruct((_N, _DI), f32),
                 jax.ShapeDtypeStruct((_N, _DI), f32)],
  )(a1, ht, rinv, Wl1, bl1.reshape(1, -1), Wr1, g2.reshape(1, -1),
    be2.reshape(1, -1), Wl2, bl2.reshape(1, -1), Wr2)

  a2 = _agg2(p, src, dst, z128)
  out = pl.pallas_call(
      _tc_c_body,
      out_shape=jax.ShapeDtypeStruct((_N, _DI), f32),
  )(a2, rinv, r)
  return out


# speedup vs baseline: 10.7290x; 2.2306x over previous
import functools

import jax
import jax.numpy as jnp
from jax import lax
from jax.experimental import pallas as pl
from jax.experimental.pallas import tpu as pltpu
from jax.experimental.pallas import tpu_sc as plsc

_N = 10000
_E = 320000
_DI = 128
_DH = 256
_DA = 144
_EPS = 1e-5

_CORES = 2
_TILES = 16
_K = 80
_NP = 10240
_RPT = _NP // _TILES


def _make_agg(D, mode):
  per_tile = _E // (_CORES * _TILES) if mode == "edge" else _E // _TILES
  chunks = per_tile // _K
  assert chunks * _K == per_tile
  pairs, tail = divmod(chunks, 2)

  mesh = plsc.VectorSubcoreMesh(core_axis_name="c", subcore_axis_name="s")

  @functools.partial(
      pl.kernel,
      mesh=mesh,
      out_type=jax.ShapeDtypeStruct((_CORES, _NP, D), jnp.float32),
      scratch_types=[
          pltpu.VMEM_SHARED((_NP, D), jnp.float32),
          pltpu.VMEM((chunks, _K), jnp.int32),
          pltpu.VMEM((_K,), jnp.int32),
          pltpu.VMEM((_K,), jnp.int32),
          pltpu.VMEM((_K, D), jnp.float32),
          pltpu.VMEM((_K, D), jnp.float32),
          pltpu.SemaphoreType.DMA,
          pltpu.SemaphoreType.DMA,
          pltpu.SemaphoreType.DMA,
          pltpu.SemaphoreType.DMA,
      ],
      compiler_params=pltpu.CompilerParams(use_tc_tiling_on_sc=False),
  )
  def agg(table, src_hbm, dst_hbm, zeros_hbm, out_hbm, acc, dstv,
          src_a, src_b, rows_a, rows_b, isem_a, isem_b, gsem_a, gsem_b):
    c = lax.axis_index("c")
    s = lax.axis_index("s")
    if mode == "edge":
      srow = (c * _TILES + s) * chunks
      drow = srow
    else:
      srow = c * (_E // _K) + s * chunks
      drow = s * chunks
    pltpu.sync_copy(dst_hbm.at[pl.ds(drow, chunks)], dstv)
    pltpu.sync_copy(zeros_hbm.at[pl.ds(s * _RPT, _RPT)],
                    acc.at[pl.ds(s * _RPT, _RPT)])
    plsc.subcore_barrier()

    def fire_src(i, buf, sem):
      pltpu.async_copy(src_hbm.at[srow + i], buf, sem)

    def wait_src(buf, sem):
      pltpu.make_async_copy(src_hbm.at[srow], buf, sem).wait()

    def fire_gather(buf, rows, sem):
      pltpu.async_copy(table.at[buf], rows, sem)

    def wait_gather(buf, rows, sem):
      pltpu.make_async_copy(table.at[buf], rows, sem).wait()

    def scatter(i, rows):
      pltpu.sync_copy(rows, acc.at[dstv.at[i]], add=True)

    fire_src(0, src_a, isem_a)
    fire_src(1, src_b, isem_b)
    wait_src(src_a, isem_a)
    fire_gather(src_a, rows_a, gsem_a)
    wait_src(src_b, isem_b)
    fire_gather(src_b, rows_b, gsem_b)

    def pair(t, carry):
      i0 = 2 * t
      i1 = i0 + 1
      wait_gather(src_a, rows_a, gsem_a)

      @pl.when(i0 + 2 < chunks)
      def _():
        fire_src(i0 + 2, src_a, isem_a)
      scatter(i0, rows_a)
      wait_gather(src_b, rows_b, gsem_b)

      @pl.when(i1 + 2 < chunks)
      def _():
        fire_src(i1 + 2, src_b, isem_b)

      @pl.when(i0 + 2 < chunks)
      def _():
        wait_src(src_a, isem_a)
        fire_gather(src_a, rows_a, gsem_a)
      scatter(i1, rows_b)

      @pl.when(i1 + 2 < chunks)
      def _():
        wait_src(src_b, isem_b)
        fire_gather(src_b, rows_b, gsem_b)

      return carry

    lax.fori_loop(0, pairs, pair, 0)
    if tail:
      wait_gather(src_a, rows_a, gsem_a)
      scatter(chunks - 1, rows_a)

    plsc.subcore_barrier()
    pltpu.sync_copy(acc.at[pl.ds(s * _RPT, _RPT)],
                    out_hbm.at[c, pl.ds(s * _RPT, _RPT)])

  return agg


_agg0 = _make_agg(_DA, "edge")
_agg1 = _make_agg(_DI, "feat")
_agg2 = _make_agg(_DI, "edge")


def _bn_relu(h, g, be):
  mu = jnp.mean(h, 0, keepdims=True)
  var = jnp.mean((h - mu) ** 2, 0, keepdims=True)
  return jnp.maximum((h - mu) * lax.rsqrt(var + _EPS) * g + be, 0.0)


def _mm_t(a, w):
  return lax.dot_general(a, w, (((1,), (1,)), ((), ())),
                         preferred_element_type=jnp.float32)


def _tc_a_body(a0_ref, x_ref, wl_ref, bl_ref, wr_ref, g_ref, be_ref,
               ht_ref, rinv_ref):
  a = a0_ref[...]
  s = a[0, :_N] + a[1, :_N]
  deg = s[:, _DI:_DI + 1]
  rinv = 1.0 / jnp.maximum(deg, 1.0)
  h = (_mm_t(s[:, :_DI] * rinv, wl_ref[...]) + bl_ref[...]
       + _mm_t(x_ref[...], wr_ref[...]))
  h1 = _bn_relu(h, g_ref[...], be_ref[...])
  ht_ref[...] = jnp.concatenate([h1[:, :_DI], h1[:, _DI:]], 0)
  rinv_ref[...] = rinv


def _tc_b_body(a1_ref, ht_ref, rinv_ref, wl_ref, bl_ref, wr_ref, g_ref,
               be_ref, wl2_ref, bl2_ref, wr2_ref, p_ref, r_ref):
  a = a1_ref[...]
  ht = ht_ref[...]
  rinv = rinv_ref[...]
  agg = jnp.concatenate([a[0, :_N], a[1, :_N]], 1) * rinv
  h1 = jnp.concatenate([ht[:_N], ht[_N:]], 1)
  h = _mm_t(agg, wl_ref[...]) + bl_ref[...] + _mm_t(h1, wr_ref[...])
  h2 = _bn_relu(h, g_ref[...], be_ref[...])
  p_ref[...] = _mm_t(h2, wl2_ref[...])
  r_ref[...] = _mm_t(h2, wr2_ref[...]) + bl2_ref[...]


def _tc_c_body(a2_ref, rinv_ref, r_ref, out_ref):
  a = a2_ref[...]
  h = (a[0, :_N] + a[1, :_N]) * rinv_ref[...] + r_ref[...]
  m = jnp.max(h, -1, keepdims=True)
  lse = jnp.log(jnp.sum(jnp.exp(h - m), -1, keepdims=True)) + m
  out_ref[...] = h - lse


def kernel(x, edge_index, Wl0, bl0, Wr0, Wl1, bl1, Wr1, Wl2, bl2, Wr2,
           g1, be1, g2, be2):
  src = edge_index[0]
  dst = edge_index[1]
  f32 = jnp.float32
  x_aug = jnp.concatenate(
      [x, jnp.ones((_N, 1), f32), jnp.zeros((_N, _DA - _DI - 1), f32)], 1)
  z144 = jnp.zeros((_NP, _DA), f32)
  z128 = jnp.zeros((_NP, _DI), f32)
  src2 = src.reshape(_E // _K, _K)
  dst2 = dst.reshape(_E // _K, _K)
  src_f = jnp.concatenate([src, src + _N]).reshape(2 * _E // _K, _K)

  a0 = _agg0(x_aug, src2, dst2, z144)
  ht, rinv = pl.pallas_call(
      _tc_a_body,
      out_shape=[jax.ShapeDtypeStruct((2 * _N, _DI), f32),
                 jax.ShapeDtypeStruct((_N, 1), f32)],
  )(a0, x, Wl0, bl0.reshape(1, -1), Wr0, g1.reshape(1, -1),
    be1.reshape(1, -1))

  a1 = _agg1(ht, src_f, dst2, z128)
  p, r = pl.pallas_call(
      _tc_b_body,
      out_shape=[jax.ShapeDtypeStruct((_N, _DI), f32),
                 jax.ShapeDtypeStruct((_N, _DI), f32)],
  )(a1, ht, rinv, Wl1, bl1.reshape(1, -1), Wr1, g2.reshape(1, -1),
    be2.reshape(1, -1), Wl2, bl2.reshape(1, -1), Wr2)

  a2 = _agg2(p, src2, dst2, z128)
  out = pl.pallas_call(
      _tc_c_body,
      out_shape=jax.ShapeDtypeStruct((_N, _DI), f32),
  )(a2, rinv, r)
  return out


# speedup vs baseline: 11.7594x; 1.0960x over previous
import functools

import jax
import jax.numpy as jnp
from jax import lax
from jax.experimental import pallas as pl
from jax.experimental.pallas import tpu as pltpu
from jax.experimental.pallas import tpu_sc as plsc

_N = 10000
_E = 320000
_DI = 128
_DH = 256
_DA = 144
_EPS = 1e-5

_CORES = 2
_TILES = 16
_K = 128
_NP = 10112
_RPT = _NP // _TILES


def _make_agg(D, mode):
  per_tile = _E // (_CORES * _TILES) if mode == "edge" else _E // _TILES
  chunks, tail_k = divmod(per_tile, _K)
  assert chunks % 2 == 0 and tail_k % 8 == 0 and tail_k > 0
  pairs = chunks // 2

  mesh = plsc.VectorSubcoreMesh(core_axis_name="c", subcore_axis_name="s")

  @functools.partial(
      pl.kernel,
      mesh=mesh,
      out_type=jax.ShapeDtypeStruct((_CORES, _NP, D), jnp.float32),
      scratch_types=[
          pltpu.VMEM_SHARED((_NP, D), jnp.float32),
          pltpu.VMEM((_K,), jnp.int32),
          pltpu.VMEM((_K,), jnp.int32),
          pltpu.VMEM((_K,), jnp.int32),
          pltpu.VMEM((_K,), jnp.int32),
          pltpu.VMEM((_K, D), jnp.float32),
          pltpu.VMEM((_K, D), jnp.float32),
          pltpu.VMEM((tail_k,), jnp.int32),
          pltpu.VMEM((tail_k,), jnp.int32),
          pltpu.VMEM((tail_k, D), jnp.float32),
          pltpu.SemaphoreType.DMA,
          pltpu.SemaphoreType.DMA,
          pltpu.SemaphoreType.DMA,
          pltpu.SemaphoreType.DMA,
          pltpu.SemaphoreType.DMA,
          pltpu.SemaphoreType.DMA,
          pltpu.SemaphoreType.DMA,
      ],
      compiler_params=pltpu.CompilerParams(use_tc_tiling_on_sc=False),
  )
  def agg(table, src_hbm, dst_hbm, zeros_hbm, out_hbm, acc,
          src_a, src_b, dst_a, dst_b, rows_a, rows_b,
          src_t, dst_t, rows_t,
          isem_a, isem_b, dsem_a, dsem_b, gsem_a, gsem_b, tsem):
    c = lax.axis_index("c")
    s = lax.axis_index("s")
    if mode == "edge":
      base = (c * _TILES + s) * per_tile
      src_off = base
      dst_off = base
    else:
      dst_off = s * per_tile
      src_off = c * _E + dst_off

    def fire_idx(i, sbuf, dbuf, ssem, dsem, width):
      pltpu.async_copy(src_hbm.at[pl.ds(src_off + i * _K, width)], sbuf, ssem)
      pltpu.async_copy(dst_hbm.at[pl.ds(dst_off + i * _K, width)], dbuf, dsem)

    def wait_one(hbm, buf, sem):
      pltpu.make_async_copy(hbm.at[pl.ds(0, buf.shape[0])], buf, sem).wait()

    def fire_gather(buf, rows, sem):
      pltpu.async_copy(table.at[buf], rows, sem)

    def wait_gather(buf, rows, sem):
      pltpu.make_async_copy(table.at[buf], rows, sem).wait()

    def scatter(dbuf, rows):
      pltpu.sync_copy(rows, acc.at[dbuf], add=True)

    fire_idx(0, src_a, dst_a, isem_a, dsem_a, _K)
    fire_idx(1, src_b, dst_b, isem_b, dsem_b, _K)
    pltpu.async_copy(src_hbm.at[pl.ds(src_off + chunks * _K, tail_k)],
                     src_t, tsem)
    pltpu.async_copy(dst_hbm.at[pl.ds(dst_off + chunks * _K, tail_k)],
                     dst_t, tsem)
    for j in range(_RPT // _K):
      pltpu.sync_copy(zeros_hbm, acc.at[pl.ds(s * _RPT + j * _K, _K)])
    rem = _RPT % _K
    if rem:
      pltpu.sync_copy(zeros_hbm.at[pl.ds(0, rem)],
                      acc.at[pl.ds(s * _RPT + (_RPT // _K) * _K, rem)])
    wait_one(src_hbm, src_a, isem_a)
    fire_gather(src_a, rows_a, gsem_a)
    wait_one(src_hbm, src_b, isem_b)
    fire_gather(src_b, rows_b, gsem_b)
    plsc.subcore_barrier()

    def pair(t, carry):
      i0 = 2 * t
      i1 = i0 + 1
      wait_gather(src_a, rows_a, gsem_a)

      @pl.when(i0 + 2 < chunks)
      def _():
        pltpu.async_copy(src_hbm.at[pl.ds(src_off + (i0 + 2) * _K, _K)],
                         src_a, isem_a)
      wait_one(dst_hbm, dst_a, dsem_a)
      scatter(dst_a, rows_a)

      @pl.when(i0 + 2 < chunks)
      def _():
        pltpu.async_copy(dst_hbm.at[pl.ds(dst_off + (i0 + 2) * _K, _K)],
                         dst_a, dsem_a)
        wait_one(src_hbm, src_a, isem_a)
        fire_gather(src_a, rows_a, gsem_a)

      wait_gather(src_b, rows_b, gsem_b)

      @pl.when(i1 + 2 < chunks)
      def _():
        pltpu.async_copy(src_hbm.at[pl.ds(src_off + (i1 + 2) * _K, _K)],
                         src_b, isem_b)
      wait_one(dst_hbm, dst_b, dsem_b)
      scatter(dst_b, rows_b)

      @pl.when(i1 + 2 < chunks)
      def _():
        pltpu.async_copy(dst_hbm.at[pl.ds(dst_off + (i1 + 2) * _K, _K)],
                         dst_b, dsem_b)
        wait_one(src_hbm, src_b, isem_b)
        fire_gather(src_b, rows_b, gsem_b)

      return carry

    lax.fori_loop(0, pairs, pair, 0)

    wait_one(src_hbm, src_t, tsem)
    wait_one(dst_hbm, dst_t, tsem)
    fire_gather(src_t, rows_t, gsem_a)
    wait_gather(src_t, rows_t, gsem_a)
    scatter(dst_t, rows_t)

    plsc.subcore_barrier()
    pltpu.sync_copy(acc.at[pl.ds(s * _RPT, _RPT)],
                    out_hbm.at[c, pl.ds(s * _RPT, _RPT)])

  return agg


_agg0 = _make_agg(_DA, "edge")
_agg1 = _make_agg(_DI, "feat")
_agg2 = _make_agg(_DI, "edge")


def _bn_relu(h, g, be):
  mu = jnp.mean(h, 0, keepdims=True)
  var = jnp.mean((h - mu) ** 2, 0, keepdims=True)
  return jnp.maximum((h - mu) * lax.rsqrt(var + _EPS) * g + be, 0.0)


def _mm_t(a, w):
  return lax.dot_general(a, w, (((1,), (1,)), ((), ())),
                         preferred_element_type=jnp.float32)


def _tc_a_body(a0_ref, x_ref, wl_ref, bl_ref, wr_ref, g_ref, be_ref,
               ht_ref, rinv_ref):
  a = a0_ref[...]
  s = a[0, :_N] + a[1, :_N]
  deg = s[:, _DI:_DI + 1]
  rinv = 1.0 / jnp.maximum(deg, 1.0)
  h = (_mm_t(s[:, :_DI] * rinv, wl_ref[...]) + bl_ref[...]
       + _mm_t(x_ref[...], wr_ref[...]))
  h1 = _bn_relu(h, g_ref[...], be_ref[...])
  ht_ref[...] = jnp.concatenate([h1[:, :_DI], h1[:, _DI:]], 0)
  rinv_ref[...] = rinv


def _tc_b_body(a1_ref, ht_ref, rinv_ref, wl_ref, bl_ref, wr_ref, g_ref,
               be_ref, wl2_ref, bl2_ref, wr2_ref, p_ref, r_ref):
  a = a1_ref[...]
  ht = ht_ref[...]
  rinv = rinv_ref[...]
  agg = jnp.concatenate([a[0, :_N], a[1, :_N]], 1) * rinv
  h1 = jnp.concatenate([ht[:_N], ht[_N:]], 1)
  h = _mm_t(agg, wl_ref[...]) + bl_ref[...] + _mm_t(h1, wr_ref[...])
  h2 = _bn_relu(h, g_ref[...], be_ref[...])
  p_ref[...] = _mm_t(h2, wl2_ref[...])
  r_ref[...] = _mm_t(h2, wr2_ref[...]) + bl2_ref[...]


def _tc_c_body(a2_ref, rinv_ref, r_ref, out_ref):
  a = a2_ref[...]
  h = (a[0, :_N] + a[1, :_N]) * rinv_ref[...] + r_ref[...]
  m = jnp.max(h, -1, keepdims=True)
  lse = jnp.log(jnp.sum(jnp.exp(h - m), -1, keepdims=True)) + m
  out_ref[...] = h - lse


def kernel(x, edge_index, Wl0, bl0, Wr0, Wl1, bl1, Wr1, Wl2, bl2, Wr2,
           g1, be1, g2, be2):
  src = edge_index[0]
  dst = edge_index[1]
  f32 = jnp.float32
  x_aug = jnp.concatenate(
      [x, jnp.ones((_N, 1), f32), jnp.zeros((_N, _DA - _DI - 1), f32)], 1)
  z144 = jnp.zeros((_K, _DA), f32)
  z128 = jnp.zeros((_K, _DI), f32)
  src_f = jnp.concatenate([src, src + _N])

  a0 = _agg0(x_aug, src, dst, z144)
  ht, rinv = pl.pallas_call(
      _tc_a_body,
      out_shape=[jax.ShapeDtypeStruct((2 * _N, _DI), f32),
                 jax.ShapeDtypeStruct((_N, 1), f32)],
  )(a0, x, Wl0, bl0.reshape(1, -1), Wr0, g1.reshape(1, -1),
    be1.reshape(1, -1))

  a1 = _agg1(ht, src_f, dst, z128)
  p, r = pl.pallas_call(
      _tc_b_body,
      out_shape=[jax.ShapeDtypeStruct((_N, _DI), f32),
                 jax.ShapeDtypeStruct((_N, _DI), f32)],
  )(a1, ht, rinv, Wl1, bl1.reshape(1, -1), Wr1, g2.reshape(1, -1),
    be2.reshape(1, -1), Wl2, bl2.reshape(1, -1), Wr2)

  a2 = _agg2(p, src, dst, z128)
  out = pl.pallas_call(
      _tc_c_body,
      out_shape=jax.ShapeDtypeStruct((_N, _DI), f32),
  )(a2, rinv, r)
  return out
